# Initial kernel scaffold; baseline (speedup 1.0000x reference)
import jax
import jax.numpy as jnp
from jax import lax
from jax.experimental import pallas as pl
from jax.experimental.pallas import tpu as pltpu
from jax.experimental.pallas import tpu_sc as plsc

VOCAB = 100000
DIM = 8
KS = 256
SUBDIM = 16
ORIG = DIM * SUBDIM

NC, NS, LANES = 2, 16, 16
NW = NC * NS
N_TOK = 4096 * 50
C = 128
TPW = N_TOK // NW
CHUNKS = TPW // C


def _pq_decode_body(idx_hbm, indexes_hbm, vectors_hbm, out_hbm,
                    idx_v, codes_v, flat_v, out_v, sem):
    wid = lax.axis_index("s") * NC + lax.axis_index("c")
    base = wid * TPW

    iot = lax.broadcasted_iota(jnp.int32, (16,), 0)
    row_pat = iot >> 3
    col_pat = iot & 7
    off_pat = col_pat << 8

    @pl.loop(0, CHUNKS)
    def _chunk(ci):
        t0 = base + ci * C
        pltpu.sync_copy(idx_hbm.at[pl.ds(t0, C)], idx_v)
        pltpu.async_copy(indexes_hbm.at[idx_v], codes_v, sem).wait()

        @pl.loop(0, C // 2)
        def _grp(g):
            codes16 = plsc.load_gather(codes_v, [row_pat + 2 * g, col_pat])
            flat_v[pl.ds(16 * g, 16)] = codes16 + off_pat

        copies = []
        for j in range(DIM):
            copies.append(pltpu.async_copy(
                vectors_hbm.at[flat_v.at[pl.ds(128 * j, 128)]],
                out_v.at[pl.ds(128 * j, 128)], sem))
        for cp in copies:
            cp.wait()

        pltpu.sync_copy(out_v, out_hbm.at[pl.ds(t0 * DIM, C * DIM)])


def kernel(idx, indexes, vectors, dims):
    del dims
    idx_flat = idx.reshape(-1)
    vec_flat = vectors.reshape(DIM * KS, SUBDIM)
    mesh = plsc.VectorSubcoreMesh(core_axis_name="c", subcore_axis_name="s")
    decode = pl.kernel(
        _pq_decode_body,
        out_type=jax.ShapeDtypeStruct((N_TOK * DIM, SUBDIM), jnp.float32),
        mesh=mesh,
        scratch_types=[
            pltpu.VMEM((C,), jnp.int32),
            pltpu.VMEM((C, DIM), jnp.int32),
            pltpu.VMEM((C * DIM,), jnp.int32),
            pltpu.VMEM((C * DIM, SUBDIM), jnp.float32),
            pltpu.SemaphoreType.DMA,
        ],
    )
    out = decode(idx_flat, indexes, vec_flat)
    return out.reshape(idx.shape + (ORIG,))


# speedup vs baseline: 37.4165x; 37.4165x over previous
import dataclasses

import jax
import jax.numpy as jnp
from jax import lax
from jax.experimental import pallas as pl
from jax.experimental.pallas import tpu as pltpu
from jax.experimental.pallas import tpu_sc as plsc

VOCAB = 100000
DIM = 8
KS = 256
SUBDIM = 16
ORIG = DIM * SUBDIM

NC, NS, LANES = 2, 16, 16
NW = NC * NS
N_TOK = 4096 * 50
C = 128
TPW = N_TOK // NW
CHUNKS = TPW // C


def _pq_decode_body(idx_hbm, indexes_hbm, vectors_hbm, out_hbm,
                    idx_v, codes_v, flat_v, out_v, sem):
    wid = lax.axis_index("s") * NC + lax.axis_index("c")
    base = wid * TPW

    iot = lax.broadcasted_iota(jnp.int32, (16,), 0)
    row_pat = iot >> 3
    col_pat = iot & 7
    off_pat = col_pat << 8

    @pl.loop(0, CHUNKS)
    def _chunk(ci):
        t0 = base + ci * C
        pltpu.sync_copy(idx_hbm.at[pl.ds(t0, C)], idx_v)
        pltpu.async_copy(indexes_hbm.at[idx_v], codes_v, sem).wait()

        @pl.loop(0, C // 2)
        def _grp(g):
            codes16 = plsc.load_gather(codes_v, [row_pat + 2 * g, col_pat])
            flat_v[pl.ds(16 * g, 16)] = codes16 + off_pat

        copies = []
        for j in range(DIM):
            copies.append(pltpu.async_copy(
                vectors_hbm.at[flat_v.at[pl.ds(128 * j, 128)]],
                out_v.at[pl.ds(128 * j, 128)], sem))
        for cp in copies:
            cp.wait()

        pltpu.sync_copy(out_v, out_hbm.at[pl.ds(t0 * DIM, C * DIM)])


def kernel(idx, indexes, vectors, dims):
    del dims
    idx_flat = idx.reshape(-1)
    vec_flat = vectors.reshape(DIM * KS, SUBDIM)
    mesh = plsc.VectorSubcoreMesh(core_axis_name="c", subcore_axis_name="s")
    cp = pltpu.CompilerParams(
        needs_layout_passes=False, use_tc_tiling_on_sc=False)
    decode = pl.kernel(
        _pq_decode_body,
        out_type=jax.ShapeDtypeStruct((N_TOK * DIM, SUBDIM), jnp.float32),
        mesh=mesh,
        scratch_types=[
            pltpu.VMEM((C,), jnp.int32),
            pltpu.VMEM((C, DIM), jnp.int32),
            pltpu.VMEM((C * DIM,), jnp.int32),
            pltpu.VMEM((C * DIM, SUBDIM), jnp.float32),
            pltpu.SemaphoreType.DMA,
        ],
        compiler_params=cp,
    )
    out = decode(idx_flat, indexes, vec_flat)
    return out.reshape(idx.shape + (ORIG,))


# speedup vs baseline: 43.8606x; 1.1722x over previous
import jax
import jax.numpy as jnp
from jax import lax
from jax.experimental import pallas as pl
from jax.experimental.pallas import tpu as pltpu
from jax.experimental.pallas import tpu_sc as plsc

VOCAB = 100000
DIM = 8
KS = 256
SUBDIM = 16
ORIG = DIM * SUBDIM

NC, NS, LANES = 2, 16, 16
NW = NC * NS
N_TOK = 4096 * 50
C = 128
TPW = N_TOK // NW
PAIRS = TPW // (2 * C)


def _pq_decode_body(idx_hbm, indexes_hbm, vectors_hbm, out_hbm,
                    idx0, idx1, codes0, codes1, flat0, flat1, out0, out1,
                    semb0, semb1, semd0, semd1, seme0, seme1):
    wid = lax.axis_index("s") * NC + lax.axis_index("c")
    base = wid * TPW
    idx_v = (idx0, idx1)
    codes_v = (codes0, codes1)
    flat_v = (flat0, flat1)
    out_v = (out0, out1)
    semb = (semb0, semb1)
    semd = (semd0, semd1)
    seme = (seme0, seme1)

    iot = lax.broadcasted_iota(jnp.int32, (16,), 0)
    row_pat = iot >> 3
    col_pat = iot & 7
    off_pat = col_pat << 8

    def fire_ab(t0, b):
        pltpu.sync_copy(idx_hbm.at[pl.ds(t0, C)], idx_v[b])
        pltpu.async_copy(indexes_hbm.at[idx_v[b]], codes_v[b], semb[b])

    def wait_b(b):
        pltpu.make_async_copy(
            indexes_hbm.at[idx_v[b]], codes_v[b], semb[b]).wait()

    def build_flat(b):
        @pl.loop(0, C // 2)
        def _grp(g):
            codes16 = plsc.load_gather(codes_v[b], [row_pat + 2 * g, col_pat])
            flat_v[b][pl.ds(16 * g, 16)] = codes16 + off_pat

    def fire_d(b):
        for j in range(DIM):
            pltpu.async_copy(
                vectors_hbm.at[flat_v[b].at[pl.ds(128 * j, 128)]],
                out_v[b].at[pl.ds(128 * j, 128)], semd[b])

    def wait_d(b):
        for j in range(DIM):
            pltpu.make_async_copy(
                vectors_hbm.at[flat_v[b].at[pl.ds(128 * j, 128)]],
                out_v[b].at[pl.ds(128 * j, 128)], semd[b]).wait()

    def fire_e(t0, b):
        pltpu.async_copy(out_v[b], out_hbm.at[pl.ds(t0 * DIM, C * DIM)], seme[b])

    def wait_e(b):
        pltpu.make_async_copy(
            out_v[b], out_hbm.at[pl.ds(0, C * DIM)], seme[b]).wait()

    fire_ab(base, 0)
    fire_ab(base + C, 1)

    @pl.loop(0, PAIRS)
    def _pair(g):
        t0 = base + (2 * g) * C
        t1 = t0 + C

        wait_b(0)
        build_flat(0)

        @pl.when(g > 0)
        def _():
            wait_e(0)
        fire_d(0)

        wait_b(1)
        build_flat(1)

        @pl.when(g > 0)
        def _():
            wait_e(1)
        fire_d(1)

        @pl.when(g < PAIRS - 1)
        def _():
            fire_ab(t0 + 2 * C, 0)
            fire_ab(t0 + 3 * C, 1)

        wait_d(0)
        fire_e(t0, 0)
        wait_d(1)
        fire_e(t1, 1)

    wait_e(0)
    wait_e(1)


def kernel(idx, indexes, vectors, dims):
    del dims
    idx_flat = idx.reshape(-1)
    vec_flat = vectors.reshape(DIM * KS, SUBDIM)
    mesh = plsc.VectorSubcoreMesh(core_axis_name="c", subcore_axis_name="s")
    cp = pltpu.CompilerParams(
        needs_layout_passes=False, use_tc_tiling_on_sc=False)
    decode = pl.kernel(
        _pq_decode_body,
        out_type=jax.ShapeDtypeStruct((N_TOK * DIM, SUBDIM), jnp.float32),
        mesh=mesh,
        scratch_types=[
            pltpu.VMEM((C,), jnp.int32),
            pltpu.VMEM((C,), jnp.int32),
            pltpu.VMEM((C, DIM), jnp.int32),
            pltpu.VMEM((C, DIM), jnp.int32),
            pltpu.VMEM((C * DIM,), jnp.int32),
            pltpu.VMEM((C * DIM,), jnp.int32),
            pltpu.VMEM((C * DIM, SUBDIM), jnp.float32),
            pltpu.VMEM((C * DIM, SUBDIM), jnp.float32),
            pltpu.SemaphoreType.DMA,
            pltpu.SemaphoreType.DMA,
            pltpu.SemaphoreType.DMA,
            pltpu.SemaphoreType.DMA,
            pltpu.SemaphoreType.DMA,
            pltpu.SemaphoreType.DMA,
        ],
        compiler_params=cp,
    )
    out = decode(idx_flat, indexes, vec_flat)
    return out.reshape(idx.shape + (ORIG,))
